# Initial kernel scaffold; baseline (speedup 1.0000x reference)
import jax
import jax.numpy as jnp
from jax.experimental import pallas as pl


def kernel(x, edge_index, adj_values, noise, W_l, b_l, W_r, b_r, W_a, b_a):
    raise NotImplementedError("write your pallas kernel here")


# speedup vs baseline: 20.8536x; 20.8536x over previous
import functools

import jax
import jax.numpy as jnp
from jax import lax
from jax.experimental import pallas as pl
from jax.experimental.pallas import tpu as pltpu
from jax.experimental.pallas import tpu_sc as plsc

GAMMA = -0.1
ZETA = 1.1
LANES = 16


def _node_scores_body(x_ref, wlT_ref, wrT_ref, bl_ref, br_ref, al_ref, ar_ref,
                      ba_ref, sl_ref, sr_ref):
    x = x_ref[...]
    hl = jnp.maximum(
        jnp.dot(x, wlT_ref[...], preferred_element_type=jnp.float32)
        + bl_ref[...], 0.0)
    sl_ref[...] = jnp.sum(hl * al_ref[...], axis=1, keepdims=True) + ba_ref[0, 0]
    hr = jnp.maximum(
        jnp.dot(x, wrT_ref[...], preferred_element_type=jnp.float32)
        + br_ref[...], 0.0)
    sr_ref[...] = jnp.sum(hr * ar_ref[...], axis=1, keepdims=True)


def _node_scores(x, W_l, b_l, W_r, b_r, W_a, b_a):
    n, d = x.shape
    h = W_l.shape[0]
    blk = 2000
    assert n % blk == 0
    wlT = W_l.T
    wrT = W_r.T
    al = W_a[:, :h].reshape(1, h)
    ar = W_a[:, h:].reshape(1, h)
    bl = b_l.reshape(1, h)
    br = b_r.reshape(1, h)
    ba = b_a.reshape(1, 1)
    full = lambda i: (0, 0)
    sl, sr = pl.pallas_call(
        _node_scores_body,
        grid=(n // blk,),
        in_specs=[
            pl.BlockSpec((blk, d), lambda i: (i, 0)),
            pl.BlockSpec((d, h), full),
            pl.BlockSpec((d, h), full),
            pl.BlockSpec((1, h), full),
            pl.BlockSpec((1, h), full),
            pl.BlockSpec((1, h), full),
            pl.BlockSpec((1, h), full),
            pl.BlockSpec((1, 1), full),
        ],
        out_specs=[
            pl.BlockSpec((blk, 1), lambda i: (i, 0)),
            pl.BlockSpec((blk, 1), lambda i: (i, 0)),
        ],
        out_shape=[
            jax.ShapeDtypeStruct((n, 1), jnp.float32),
            jax.ShapeDtypeStruct((n, 1), jnp.float32),
        ],
    )(x, wlT, wrT, bl, br, al, ar, ba)
    return sl.reshape(n), sr.reshape(n)


def _edge_gate(sl, sr, row, col, noise, adj_values):
    n = sl.shape[0]
    e = noise.shape[0]
    info = plsc.get_sparse_core_info()
    nc, ns = info.num_cores, info.num_subcores
    nw = nc * ns
    assert e % (nw * LANES) == 0
    epw = e // nw

    mesh = plsc.VectorSubcoreMesh(core_axis_name="c", subcore_axis_name="s")

    @functools.partial(
        pl.kernel,
        out_type=jax.ShapeDtypeStruct((e,), jnp.float32),
        mesh=mesh,
        compiler_params=pltpu.CompilerParams(needs_layout_passes=False),
        scratch_types=[
            pltpu.VMEM((n,), jnp.float32),
            pltpu.VMEM((n,), jnp.float32),
            pltpu.VMEM((epw,), jnp.int32),
            pltpu.VMEM((epw,), jnp.int32),
            pltpu.VMEM((epw,), jnp.float32),
            pltpu.VMEM((epw,), jnp.float32),
            pltpu.VMEM((epw,), jnp.float32),
        ],
    )
    def run(sl_hbm, sr_hbm, row_hbm, col_hbm, noise_hbm, adj_hbm, out_hbm,
            sl_v, sr_v, row_v, col_v, noise_v, adj_v, out_v):
        wid = lax.axis_index("s") * nc + lax.axis_index("c")
        base = pl.multiple_of(wid * epw, 8)
        pltpu.sync_copy(sl_hbm, sl_v)
        pltpu.sync_copy(sr_hbm, sr_v)
        pltpu.sync_copy(row_hbm.at[pl.ds(base, epw)], row_v)
        pltpu.sync_copy(col_hbm.at[pl.ds(base, epw)], col_v)
        pltpu.sync_copy(noise_hbm.at[pl.ds(base, epw)], noise_v)
        pltpu.sync_copy(adj_hbm.at[pl.ds(base, epw)], adj_v)

        def body(i, carry):
            off = i * LANES
            r = row_v[pl.ds(off, LANES)]
            c = col_v[pl.ds(off, LANES)]
            a = plsc.load_gather(sl_v, [r])
            b = plsc.load_gather(sr_v, [c])
            u = noise_v[pl.ds(off, LANES)]
            t = jnp.exp(-(a + b))
            gate = u / (u + (1.0 - u) * t)
            m = jnp.minimum(jnp.maximum(gate * (ZETA - GAMMA) + GAMMA, 0.0), 1.0)
            out_v[pl.ds(off, LANES)] = adj_v[pl.ds(off, LANES)] * m
            return carry

        lax.fori_loop(0, epw // LANES, body, 0)
        pltpu.sync_copy(out_v, out_hbm.at[pl.ds(base, epw)])

    return run(sl, sr, row, col, noise, adj_values)


def kernel(x, edge_index, adj_values, noise, W_l, b_l, W_r, b_r, W_a, b_a):
    sl, sr = _node_scores(x, W_l, b_l, W_r, b_r, W_a, b_a)
    return _edge_gate(sl, sr, edge_index[0], edge_index[1], noise, adj_values)


# speedup vs baseline: 29.7793x; 1.4280x over previous
import functools

import jax
import jax.numpy as jnp
from jax import lax
from jax.experimental import pallas as pl
from jax.experimental.pallas import tpu as pltpu
from jax.experimental.pallas import tpu_sc as plsc

GAMMA = -0.1
ZETA = 1.1
LANES = 16


def _node_scores_body(x_ref, wlT_ref, wrT_ref, bl_ref, br_ref, al_ref, ar_ref,
                      ba_ref, sl_ref, sr_ref):
    x = x_ref[...]
    hl = jnp.maximum(
        jnp.dot(x, wlT_ref[...], preferred_element_type=jnp.float32)
        + bl_ref[...], 0.0)
    sl_ref[...] = jnp.sum(hl * al_ref[...], axis=1, keepdims=True) + ba_ref[0, 0]
    hr = jnp.maximum(
        jnp.dot(x, wrT_ref[...], preferred_element_type=jnp.float32)
        + br_ref[...], 0.0)
    sr_ref[...] = jnp.sum(hr * ar_ref[...], axis=1, keepdims=True)


def _node_scores(x, W_l, b_l, W_r, b_r, W_a, b_a):
    n, d = x.shape
    h = W_l.shape[0]
    blk = 2000
    assert n % blk == 0
    wlT = W_l.T
    wrT = W_r.T
    al = W_a[:, :h].reshape(1, h)
    ar = W_a[:, h:].reshape(1, h)
    bl = b_l.reshape(1, h)
    br = b_r.reshape(1, h)
    ba = b_a.reshape(1, 1)
    full = lambda i: (0, 0)
    sl, sr = pl.pallas_call(
        _node_scores_body,
        grid=(n // blk,),
        in_specs=[
            pl.BlockSpec((blk, d), lambda i: (i, 0)),
            pl.BlockSpec((d, h), full),
            pl.BlockSpec((d, h), full),
            pl.BlockSpec((1, h), full),
            pl.BlockSpec((1, h), full),
            pl.BlockSpec((1, h), full),
            pl.BlockSpec((1, h), full),
            pl.BlockSpec((1, 1), full),
        ],
        out_specs=[
            pl.BlockSpec((blk, 1), lambda i: (i, 0)),
            pl.BlockSpec((blk, 1), lambda i: (i, 0)),
        ],
        out_shape=[
            jax.ShapeDtypeStruct((n, 1), jnp.float32),
            jax.ShapeDtypeStruct((n, 1), jnp.float32),
        ],
    )(x, wlT, wrT, bl, br, al, ar, ba)
    return sl.reshape(n), sr.reshape(n)


def _edge_gate(sl, sr, ei_flat, noise, adj_values):
    n = sl.shape[0]
    e = noise.shape[0]
    info = plsc.get_sparse_core_info()
    nc, ns = info.num_cores, info.num_subcores
    nw = nc * ns
    assert e % (nw * LANES) == 0
    epw = e // nw

    mesh = plsc.VectorSubcoreMesh(core_axis_name="c", subcore_axis_name="s")

    @functools.partial(
        pl.kernel,
        out_type=jax.ShapeDtypeStruct((e,), jnp.float32),
        mesh=mesh,
        compiler_params=pltpu.CompilerParams(needs_layout_passes=False),
        scratch_types=[
            pltpu.VMEM((n,), jnp.float32),
            pltpu.VMEM((n,), jnp.float32),
            pltpu.VMEM((epw,), jnp.int32),
            pltpu.VMEM((epw,), jnp.int32),
            pltpu.VMEM((epw,), jnp.float32),
            pltpu.VMEM((epw,), jnp.float32),
            pltpu.VMEM((epw,), jnp.float32),
        ],
    )
    def run(sl_hbm, sr_hbm, ei_hbm, noise_hbm, adj_hbm, out_hbm,
            sl_v, sr_v, row_v, col_v, noise_v, adj_v, out_v):
        wid = lax.axis_index("s") * nc + lax.axis_index("c")
        base = pl.multiple_of(wid * epw, 8)
        pltpu.sync_copy(sl_hbm, sl_v)
        pltpu.sync_copy(sr_hbm, sr_v)
        pltpu.sync_copy(ei_hbm.at[pl.ds(base, epw)], row_v)
        pltpu.sync_copy(ei_hbm.at[pl.ds(e + base, epw)], col_v)
        pltpu.sync_copy(noise_hbm.at[pl.ds(base, epw)], noise_v)
        pltpu.sync_copy(adj_hbm.at[pl.ds(base, epw)], adj_v)

        @plsc.parallel_loop(0, epw, LANES, unroll=8)
        def _(off):
            r = row_v[pl.ds(off, LANES)]
            c = col_v[pl.ds(off, LANES)]
            a = plsc.load_gather(sl_v, [r])
            b = plsc.load_gather(sr_v, [c])
            u = noise_v[pl.ds(off, LANES)]
            t = jnp.exp(-(a + b))
            gate = u / (u + (1.0 - u) * t)
            m = jnp.minimum(jnp.maximum(gate * (ZETA - GAMMA) + GAMMA, 0.0), 1.0)
            out_v[pl.ds(off, LANES)] = adj_v[pl.ds(off, LANES)] * m

        pltpu.sync_copy(out_v, out_hbm.at[pl.ds(base, epw)])

    return run(sl, sr, ei_flat, noise, adj_values)


def kernel(x, edge_index, adj_values, noise, W_l, b_l, W_r, b_r, W_a, b_a):
    sl, sr = _node_scores(x, W_l, b_l, W_r, b_r, W_a, b_a)
    return _edge_gate(sl, sr, edge_index.reshape(-1), noise, adj_values)


# speedup vs baseline: 45.3774x; 1.5238x over previous
import functools

import jax
import jax.numpy as jnp
from jax import lax
from jax.experimental import pallas as pl
from jax.experimental.pallas import tpu as pltpu
from jax.experimental.pallas import tpu_sc as plsc

GAMMA = -0.1
ZETA = 1.1
LANES = 16
EB = 128


def _node_scores_body(x_ref, wl_ref, wr_ref, bl_ref, br_ref, wa_ref, ba_ref,
                      st_ref):
    x = x_ref[...]
    h = wl_ref.shape[0]
    dn_tt = (((1,), (1,)), ((), ()))
    dn_nn = (((1,), (0,)), ((), ()))
    gl = jnp.maximum(
        lax.dot_general(wl_ref[...], x, dn_tt,
                        preferred_element_type=jnp.float32) + bl_ref[...], 0.0)
    gr = jnp.maximum(
        lax.dot_general(wr_ref[...], x, dn_tt,
                        preferred_element_type=jnp.float32) + br_ref[...], 0.0)
    sl_row = lax.dot_general(wa_ref[:, :h], gl, dn_nn,
                             preferred_element_type=jnp.float32) + ba_ref[0, 0]
    sr_row = lax.dot_general(wa_ref[:, h:], gr, dn_nn,
                             preferred_element_type=jnp.float32)
    st_ref[...] = jnp.concatenate([sl_row, sr_row], axis=0)


def _node_scores(x, W_l, b_l, W_r, b_r, W_a, b_a):
    n, d = x.shape
    h = W_l.shape[0]
    st = pl.pallas_call(
        _node_scores_body,
        out_shape=jax.ShapeDtypeStruct((2, n), jnp.float32),
    )(x, W_l, W_r, b_l.reshape(h, 1), b_r.reshape(h, 1), W_a,
      b_a.reshape(1, 1))
    return st


def _edge_gate(st, edge_index, noise, adj_values):
    n = st.shape[1]
    e = noise.shape[0]
    info = plsc.get_sparse_core_info()
    nc, ns = info.num_cores, info.num_subcores
    nw = nc * ns
    nblk = e // EB
    assert nblk * EB == e
    per = nblk // nw
    main = per * EB
    rem = nblk - per * nw
    cap = main + (EB if rem else 0)
    assert rem <= nw

    mesh = plsc.VectorSubcoreMesh(core_axis_name="c", subcore_axis_name="s")

    @functools.partial(
        pl.kernel,
        out_type=jax.ShapeDtypeStruct((e,), jnp.float32),
        mesh=mesh,
        compiler_params=pltpu.CompilerParams(needs_layout_passes=False),
        scratch_types=[
            pltpu.VMEM((2, n), jnp.float32),
            pltpu.VMEM((2, cap), jnp.int32),
            pltpu.VMEM((cap,), jnp.float32),
            pltpu.VMEM((cap,), jnp.float32),
            pltpu.VMEM((cap,), jnp.float32),
        ],
    )
    def run(st_hbm, ei_hbm, noise_hbm, adj_hbm, out_hbm,
            st_v, ei_v, noise_v, adj_v, out_v):
        wid = lax.axis_index("s") * nc + lax.axis_index("c")
        c0 = pl.multiple_of(wid * main, EB)
        x0 = pl.multiple_of(nw * main + wid * EB, EB)
        pltpu.sync_copy(st_hbm, st_v)
        pltpu.sync_copy(ei_hbm.at[:, pl.ds(c0, main)], ei_v.at[:, pl.ds(0, main)])
        pltpu.sync_copy(noise_hbm.at[pl.ds(c0, main)], noise_v.at[pl.ds(0, main)])
        pltpu.sync_copy(adj_hbm.at[pl.ds(c0, main)], adj_v.at[pl.ds(0, main)])

        @pl.when(wid < rem)
        def _():
            pltpu.sync_copy(ei_hbm.at[:, pl.ds(x0, EB)],
                            ei_v.at[:, pl.ds(main, EB)])
            pltpu.sync_copy(noise_hbm.at[pl.ds(x0, EB)],
                            noise_v.at[pl.ds(main, EB)])
            pltpu.sync_copy(adj_hbm.at[pl.ds(x0, EB)],
                            adj_v.at[pl.ds(main, EB)])

        zero16 = jnp.zeros((LANES,), jnp.int32)
        one16 = jnp.ones((LANES,), jnp.int32)

        def gate_at(off):
            r = ei_v[0, pl.ds(off, LANES)]
            c = ei_v[1, pl.ds(off, LANES)]
            a = plsc.load_gather(st_v, [zero16, r])
            b = plsc.load_gather(st_v, [one16, c])
            u = noise_v[pl.ds(off, LANES)]
            t = jnp.exp(-(a + b))
            gate = u / (u + (1.0 - u) * t)
            m = jnp.minimum(jnp.maximum(gate * (ZETA - GAMMA) + GAMMA, 0.0), 1.0)
            out_v[pl.ds(off, LANES)] = adj_v[pl.ds(off, LANES)] * m

        plsc.parallel_loop(0, main, LANES, unroll=8)(gate_at)

        @pl.when(wid < rem)
        def _():
            plsc.parallel_loop(main, main + EB, LANES, unroll=8)(gate_at)

        pltpu.sync_copy(out_v.at[pl.ds(0, main)], out_hbm.at[pl.ds(c0, main)])

        @pl.when(wid < rem)
        def _():
            pltpu.sync_copy(out_v.at[pl.ds(main, EB)],
                            out_hbm.at[pl.ds(x0, EB)])

    return run(st, edge_index, noise, adj_values)


def kernel(x, edge_index, adj_values, noise, W_l, b_l, W_r, b_r, W_a, b_a):
    st = _node_scores(x, W_l, b_l, W_r, b_r, W_a, b_a)
    return _edge_gate(st, edge_index, noise, adj_values)


# speedup vs baseline: 53.7165x; 1.1838x over previous
import functools

import jax
import jax.numpy as jnp
from jax import lax
from jax.experimental import pallas as pl
from jax.experimental.pallas import tpu as pltpu
from jax.experimental.pallas import tpu_sc as plsc

GAMMA = -0.1
ZETA = 1.1
LANES = 16
EB = 128


def _node_scores_body(x_ref, wl_ref, wr_ref, bl_ref, br_ref, wa_ref, ba_ref,
                      st_ref):
    x = x_ref[...]
    h = wl_ref.shape[0]
    dn_tt = (((1,), (1,)), ((), ()))
    dn_nn = (((1,), (0,)), ((), ()))
    bl = lax.broadcast_in_dim(bl_ref[...], (h, 1), (0,))
    br = lax.broadcast_in_dim(br_ref[...], (h, 1), (0,))
    gl = jnp.maximum(
        lax.dot_general(wl_ref[...], x, dn_tt,
                        preferred_element_type=jnp.float32) + bl, 0.0)
    gr = jnp.maximum(
        lax.dot_general(wr_ref[...], x, dn_tt,
                        preferred_element_type=jnp.float32) + br, 0.0)
    sl_row = lax.dot_general(wa_ref[:, :h], gl, dn_nn,
                             preferred_element_type=jnp.float32) + ba_ref[0]
    sr_row = lax.dot_general(wa_ref[:, h:], gr, dn_nn,
                             preferred_element_type=jnp.float32)
    st_ref[...] = jnp.concatenate([sl_row, sr_row], axis=0)


def _node_scores(x, W_l, b_l, W_r, b_r, W_a, b_a):
    n, d = x.shape
    h = W_l.shape[0]
    st = pl.pallas_call(
        _node_scores_body,
        out_shape=jax.ShapeDtypeStruct((2, n), jnp.float32),
    )(x, W_l, W_r, b_l, b_r, W_a, b_a)
    return st


def _edge_gate(st, edge_index, noise, adj_values):
    n = st.shape[1]
    e = noise.shape[0]
    info = plsc.get_sparse_core_info()
    nc, ns = info.num_cores, info.num_subcores
    nw = nc * ns
    nblk = e // EB
    assert nblk * EB == e
    per = nblk // nw
    main = per * EB
    rem = nblk - per * nw
    cap = main + (EB if rem else 0)
    assert rem <= nw

    mesh = plsc.VectorSubcoreMesh(core_axis_name="c", subcore_axis_name="s")

    @functools.partial(
        pl.kernel,
        out_type=jax.ShapeDtypeStruct((e,), jnp.float32),
        mesh=mesh,
        compiler_params=pltpu.CompilerParams(needs_layout_passes=False),
        scratch_types=[
            pltpu.VMEM((2, n), jnp.float32),
            pltpu.VMEM((2, cap), jnp.int32),
            pltpu.VMEM((cap,), jnp.float32),
            pltpu.VMEM((cap,), jnp.float32),
            pltpu.VMEM((cap,), jnp.float32),
            pltpu.SemaphoreType.DMA,
            pltpu.SemaphoreType.DMA,
            pltpu.SemaphoreType.DMA,
            pltpu.SemaphoreType.DMA,
        ],
    )
    def run(st_hbm, ei_hbm, noise_hbm, adj_hbm, out_hbm,
            st_v, ei_v, noise_v, adj_v, out_v,
            sem_st, sem_ei, sem_no, sem_ad):
        wid = lax.axis_index("s") * nc + lax.axis_index("c")
        c0 = pl.multiple_of(wid * main, EB)
        x0 = pl.multiple_of(nw * main + wid * EB, EB)
        cp_st = pltpu.async_copy(st_hbm, st_v, sem_st)
        cp_ei = pltpu.async_copy(ei_hbm.at[:, pl.ds(c0, main)],
                                 ei_v.at[:, pl.ds(0, main)], sem_ei)
        cp_no = pltpu.async_copy(noise_hbm.at[pl.ds(c0, main)],
                                 noise_v.at[pl.ds(0, main)], sem_no)
        cp_ad = pltpu.async_copy(adj_hbm.at[pl.ds(c0, main)],
                                 adj_v.at[pl.ds(0, main)], sem_ad)

        @pl.when(wid < rem)
        def _():
            pltpu.async_copy(ei_hbm.at[:, pl.ds(x0, EB)],
                             ei_v.at[:, pl.ds(main, EB)], sem_ei).wait()
            pltpu.async_copy(noise_hbm.at[pl.ds(x0, EB)],
                             noise_v.at[pl.ds(main, EB)], sem_no).wait()
            pltpu.async_copy(adj_hbm.at[pl.ds(x0, EB)],
                             adj_v.at[pl.ds(main, EB)], sem_ad).wait()

        cp_st.wait()
        cp_ei.wait()
        cp_no.wait()
        cp_ad.wait()

        zero16 = jnp.zeros((LANES,), jnp.int32)
        one16 = jnp.ones((LANES,), jnp.int32)

        def gate_at(off):
            r = ei_v[0, pl.ds(off, LANES)]
            c = ei_v[1, pl.ds(off, LANES)]
            a = plsc.load_gather(st_v, [zero16, r])
            b = plsc.load_gather(st_v, [one16, c])
            u = noise_v[pl.ds(off, LANES)]
            t = jnp.exp(-(a + b))
            gate = u / (u + (1.0 - u) * t)
            m = jnp.minimum(jnp.maximum(gate * (ZETA - GAMMA) + GAMMA, 0.0), 1.0)
            out_v[pl.ds(off, LANES)] = adj_v[pl.ds(off, LANES)] * m

        plsc.parallel_loop(0, main, LANES, unroll=16)(gate_at)

        @pl.when(wid < rem)
        def _():
            plsc.parallel_loop(main, main + EB, LANES, unroll=8)(gate_at)

        pltpu.sync_copy(out_v.at[pl.ds(0, main)], out_hbm.at[pl.ds(c0, main)])

        @pl.when(wid < rem)
        def _():
            pltpu.sync_copy(out_v.at[pl.ds(main, EB)],
                            out_hbm.at[pl.ds(x0, EB)])

    return run(st, edge_index, noise, adj_values)


def kernel(x, edge_index, adj_values, noise, W_l, b_l, W_r, b_r, W_a, b_a):
    st = _node_scores(x, W_l, b_l, W_r, b_r, W_a, b_a)
    return _edge_gate(st, edge_index, noise, adj_values)
